# baseline (speedup 1.0000x reference)
import functools

import jax
import jax.numpy as jnp
from jax import lax
from jax.experimental import pallas as pl
from jax.experimental.pallas import tpu as pltpu
from jax.experimental.pallas import tpu_sc as plsc

N_NODES = 10000
N_EDGES = 320000
NP = 10240
EP = 327680
IN_DIM = 128
HID = 128
HID2 = 64
NUM_CLASSES = 40

_NC = 2
_NS = 16
_NW = _NC * _NS
_EPT = EP // _NW
_ROWS_PER_TILE = NP // _NS


def _sc_mesh():
    return plsc.VectorSubcoreMesh(core_axis_name="c", subcore_axis_name="s")


@functools.cache
def _make_deg_kernel():
    KB = 16
    CH = _EPT // (KB * 128)

    @functools.partial(
        pl.kernel,
        out_type=jax.ShapeDtypeStruct((_NC, NP), jnp.float32),
        mesh=_sc_mesh(),
        scratch_types=[
            pltpu.VMEM((KB, 128), jnp.int32),
            pltpu.VMEM((KB, 128), jnp.float32),
            pltpu.VMEM((_ROWS_PER_TILE,), jnp.float32),
            pltpu.VMEM_SHARED((NP,), jnp.float32),
        ],
    )
    def deg_kernel(dst2, ew2, out, dstv, ewv, zv, acc):
        c = lax.axis_index("c")
        s = lax.axis_index("s")
        for j in range(_ROWS_PER_TILE // 16):
            zv[pl.ds(j * 16, 16)] = jnp.zeros((16,), jnp.float32)
        pltpu.sync_copy(zv, acc.at[pl.ds(s * _ROWS_PER_TILE, _ROWS_PER_TILE)])
        plsc.subcore_barrier()

        tile_row0 = (c * _NS + s) * (_EPT // 128)

        def chunk(i, carry):
            rb = tile_row0 + i * KB
            pltpu.sync_copy(dst2.at[pl.ds(rb, KB)], dstv)
            pltpu.sync_copy(ew2.at[pl.ds(rb, KB)], ewv)
            for j in range(KB):
                pltpu.sync_copy(ewv.at[j], acc.at[dstv.at[j]], add=True)
            return carry

        lax.fori_loop(0, CH, chunk, 0)
        plsc.subcore_barrier()
        pltpu.sync_copy(acc.at[pl.ds(s * _ROWS_PER_TILE, _ROWS_PER_TILE)],
                        out.at[c, pl.ds(s * _ROWS_PER_TILE, _ROWS_PER_TILE)])

    return deg_kernel


_K = 64
_NB = 4
_QC = 16
_CHT = _EPT // _K
_NQ = _CHT // _QC


@functools.cache
def _make_spmm_kernel(D):
    VR = D // 16

    @functools.partial(
        pl.kernel,
        out_type=jax.ShapeDtypeStruct((_NC, NP, D), jnp.float32),
        mesh=_sc_mesh(),
        scratch_types=[
            pltpu.VMEM((_QC, 3, _K), jnp.int32),
            pltpu.VMEM((_NB * _K, D), jnp.float32),
            pltpu.VMEM_SHARED((NP, D), jnp.float32),
            pltpu.SemaphoreType.DMA,
            pltpu.SemaphoreType.DMA,
        ],
    )
    def spmm_kernel(idx_hbm, h, out, idxall, rows, acc, sg, ss):
        c = lax.axis_index("c")
        s = lax.axis_index("s")

        def fire_gather(bo, jj):
            pltpu.async_copy(h.at[idxall.at[jj, 0]], rows.at[pl.ds(bo, _K)],
                             sg)

        def wait_gather(bo, jj):
            pltpu.make_async_copy(h.at[idxall.at[jj, 0]],
                                  rows.at[pl.ds(bo, _K)], sg).wait()

        def fire_scatter(bo, jj):
            pltpu.async_copy(rows.at[pl.ds(bo, _K)], acc.at[idxall.at[jj, 1]],
                             ss, add=True)

        def wait_scatter_any():
            pltpu.make_async_copy(rows.at[pl.ds(0, _K)],
                                  acc.at[idxall.at[0, 1]], ss).wait()

        def scale(bo, jj):
            def group(g, carry):
                w16i = idxall[jj, 2, pl.ds(g * 16, 16)]
                w16 = lax.bitcast_convert_type(w16i, jnp.float32)
                for l in range(16):
                    w = w16[l]
                    e = g * 16 + l
                    for r in range(VR):
                        rows[bo + e, pl.ds(r * 16, 16)] = (
                            rows[bo + e, pl.ds(r * 16, 16)] * w)
                return carry

            lax.fori_loop(0, _K // 16, group, 0)

        def zrow(e, carry):
            for r in range(VR):
                rows[e, pl.ds(r * 16, 16)] = jnp.zeros((16,), jnp.float32)
            return carry

        lax.fori_loop(0, _NB * _K, zrow, 0)
        off = 0
        while off < _ROWS_PER_TILE:
            n = min(_NB * _K, _ROWS_PER_TILE - off)
            pltpu.sync_copy(rows.at[pl.ds(0, n)],
                            acc.at[pl.ds(s * _ROWS_PER_TILE + off, n)])
            off += n
        plsc.subcore_barrier()

        tile_chunk0 = (c * _NS + s) * _CHT

        for q in range(_NQ):
            pltpu.sync_copy(idx_hbm.at[pl.ds(tile_chunk0 + q * _QC, _QC)],
                            idxall)
            fire_gather(0, 0)
            fire_gather(_K, 1)

            def chunk(jj, carry):
                bo = lax.bitwise_and(jj, _NB - 1) * _K
                boa = lax.bitwise_and(jj + 2, _NB - 1) * _K
                wait_gather(bo, jj)
                scale(bo, jj)
                fire_scatter(bo, jj)

                @pl.when(jj + 2 < _QC)
                def _():
                    @pl.when(jj > 0)
                    def _():
                        wait_scatter_any()

                    fire_gather(boa, jj + 2)

                return carry

            lax.fori_loop(0, _QC, chunk, 0)
            wait_scatter_any()
            wait_scatter_any()
            wait_scatter_any()

        plsc.subcore_barrier()
        pltpu.sync_copy(acc.at[pl.ds(s * _ROWS_PER_TILE, _ROWS_PER_TILE)],
                        out.at[c, pl.ds(s * _ROWS_PER_TILE, _ROWS_PER_TILE)])

    return spmm_kernel


def _l1_body(x_ref, w1_ref, degt_ref, h1p_ref, dinv_ref):
    deg = degt_ref[:, 0:1] + degt_ref[:, 1:2] + 1.0
    dinv = lax.rsqrt(deg)
    h = jnp.dot(x_ref[...], w1_ref[...],
                preferred_element_type=jnp.float32,
                precision=lax.Precision.HIGHEST)
    h1p_ref[...] = h * dinv
    dinv_ref[...] = dinv


def _l2_body(p1_ref, h1p_ref, dinv_ref, b1_ref, w2_ref, h2p_ref):
    dinv = dinv_ref[...]
    ssum = p1_ref[0] + p1_ref[1] + h1p_ref[...]
    a = jnp.maximum(ssum * dinv + b1_ref[...], 0.0)
    h2 = jnp.dot(a, w2_ref[...],
                 preferred_element_type=jnp.float32,
                 precision=lax.Precision.HIGHEST)
    h2p_ref[...] = h2 * dinv


def _l3_body(p2_ref, h2p_ref, dinv_ref, b2_ref, wc_ref, bc_ref, out_ref):
    dinv = dinv_ref[...]
    ssum = p2_ref[0] + p2_ref[1] + h2p_ref[...]
    a = jnp.maximum(ssum * dinv + b2_ref[...], 0.0)
    out_ref[...] = jnp.dot(a, wc_ref[...],
                           preferred_element_type=jnp.float32,
                           precision=lax.Precision.HIGHEST) + bc_ref[...]


def kernel(x, edge_index, edge_weight, W1, b1, W2, b2, Wc, bc):
    _deg = _make_deg_kernel()
    _spmm = _make_spmm_kernel(HID)
    f32 = jnp.float32
    src = edge_index[0].astype(jnp.int32)
    dst = edge_index[1].astype(jnp.int32)
    ew = edge_weight.astype(f32)

    padn = EP - N_EDGES
    pad_ids = jnp.arange(padn, dtype=jnp.int32)
    src_p = jnp.concatenate([src, pad_ids % N_NODES]).reshape(EP // 128, 128)
    dst_p = jnp.concatenate([dst, pad_ids % NP]).reshape(EP // 128, 128)
    ew_f = jnp.concatenate([ew, jnp.zeros((padn,), f32)])
    ew_p = ew_f.reshape(EP // 128, 128)
    ew_bits = jax.lax.bitcast_convert_type(ew_f, jnp.int32)
    idx_pack = jnp.stack(
        [src_p.reshape(EP // _K, _K), dst_p.reshape(EP // _K, _K),
         ew_bits.reshape(EP // _K, _K)], axis=1)

    x_p = jnp.zeros((NP, IN_DIM), f32).at[:N_NODES].set(x.astype(f32))
    w2_p = jnp.zeros((HID, 128), f32).at[:, :HID2].set(W2.astype(f32))
    b2_p = jnp.zeros((128,), f32).at[:HID2].set(b2.astype(f32))
    wc_p = jnp.zeros((128, 128), f32).at[:HID2, :NUM_CLASSES].set(Wc.astype(f32))
    bc_p = jnp.zeros((128,), f32).at[:NUM_CLASSES].set(bc.astype(f32))

    degp = _deg(dst_p, ew_p)
    degt = degp.T

    h1p, dinv = pl.pallas_call(
        _l1_body,
        out_shape=(jax.ShapeDtypeStruct((NP, HID), f32),
                   jax.ShapeDtypeStruct((NP, 1), f32)),
    )(x_p, W1.astype(f32), degt)

    p1 = _spmm(idx_pack, h1p)

    h2p = pl.pallas_call(
        _l2_body,
        out_shape=jax.ShapeDtypeStruct((NP, 128), f32),
    )(p1, h1p, dinv, b1.astype(f32), w2_p)

    p2 = _spmm(idx_pack, h2p)

    out = pl.pallas_call(
        _l3_body,
        out_shape=jax.ShapeDtypeStruct((NP, 128), f32),
    )(p2, h2p, dinv, b2_p, wc_p, bc_p)

    return out[:N_NODES, :NUM_CLASSES]


# speedup vs baseline: 1.0612x; 1.0612x over previous
import functools

import jax
import jax.numpy as jnp
from jax import lax
from jax.experimental import pallas as pl
from jax.experimental.pallas import tpu as pltpu
from jax.experimental.pallas import tpu_sc as plsc

N_NODES = 10000
N_EDGES = 320000
NP = 10240
EP = 327680
IN_DIM = 128
HID = 128
HID2 = 64
NUM_CLASSES = 40

_NC = 2
_NS = 16
_NW = _NC * _NS
_EPT = EP // _NW
_ROWS_PER_TILE = NP // _NS


def _sc_mesh():
    return plsc.VectorSubcoreMesh(core_axis_name="c", subcore_axis_name="s")


@functools.cache
def _make_deg_kernel():
    KB = 16
    CH = _EPT // (KB * 128)

    @functools.partial(
        pl.kernel,
        out_type=jax.ShapeDtypeStruct((_NC, NP), jnp.float32),
        mesh=_sc_mesh(),
        scratch_types=[
            pltpu.VMEM((KB, 128), jnp.int32),
            pltpu.VMEM((KB, 128), jnp.float32),
            pltpu.VMEM((_ROWS_PER_TILE,), jnp.float32),
            pltpu.VMEM_SHARED((NP,), jnp.float32),
        ],
    )
    def deg_kernel(dst2, ew2, out, dstv, ewv, zv, acc):
        c = lax.axis_index("c")
        s = lax.axis_index("s")
        for j in range(_ROWS_PER_TILE // 16):
            zv[pl.ds(j * 16, 16)] = jnp.zeros((16,), jnp.float32)
        pltpu.sync_copy(zv, acc.at[pl.ds(s * _ROWS_PER_TILE, _ROWS_PER_TILE)])
        plsc.subcore_barrier()

        tile_row0 = (c * _NS + s) * (_EPT // 128)

        def chunk(i, carry):
            rb = tile_row0 + i * KB
            pltpu.sync_copy(dst2.at[pl.ds(rb, KB)], dstv)
            pltpu.sync_copy(ew2.at[pl.ds(rb, KB)], ewv)
            for j in range(KB):
                pltpu.sync_copy(ewv.at[j], acc.at[dstv.at[j]], add=True)
            return carry

        lax.fori_loop(0, CH, chunk, 0)
        plsc.subcore_barrier()
        pltpu.sync_copy(acc.at[pl.ds(s * _ROWS_PER_TILE, _ROWS_PER_TILE)],
                        out.at[c, pl.ds(s * _ROWS_PER_TILE, _ROWS_PER_TILE)])

    return deg_kernel


_K = 64
_NB = 3
_QC = 40
_CHT = _EPT // _K
_NQ = _CHT // _QC


@functools.cache
def _make_spmm_kernel(D):
    VR = D // 16

    @functools.partial(
        pl.kernel,
        out_type=jax.ShapeDtypeStruct((_NC, NP, D), jnp.float32),
        mesh=_sc_mesh(),
        scratch_types=[
            pltpu.VMEM((_QC, 3, _K), jnp.int32),
            pltpu.VMEM((_NB * _K, D), jnp.float32),
            pltpu.VMEM_SHARED((NP, D), jnp.float32),
            pltpu.SemaphoreType.DMA,
            pltpu.SemaphoreType.DMA,
        ],
    )
    def spmm_kernel(idx_hbm, h, out, idxall, rows, acc, sg, ss):
        c = lax.axis_index("c")
        s = lax.axis_index("s")

        def fire_gather(bo, jj):
            pltpu.async_copy(h.at[idxall.at[jj, 0]], rows.at[pl.ds(bo, _K)],
                             sg)

        def wait_gather(bo, jj):
            pltpu.make_async_copy(h.at[idxall.at[jj, 0]],
                                  rows.at[pl.ds(bo, _K)], sg).wait()

        def fire_scatter(bo, jj):
            pltpu.async_copy(rows.at[pl.ds(bo, _K)], acc.at[idxall.at[jj, 1]],
                             ss, add=True)

        def wait_scatter_any():
            pltpu.make_async_copy(rows.at[pl.ds(0, _K)],
                                  acc.at[idxall.at[0, 1]], ss).wait()

        def scale(bo, jj):
            def group(g, carry):
                w16i = idxall[jj, 2, pl.ds(g * 16, 16)]
                w16 = lax.bitcast_convert_type(w16i, jnp.float32)
                for l in range(16):
                    w = w16[l]
                    e = g * 16 + l
                    for r in range(VR):
                        rows[bo + e, pl.ds(r * 16, 16)] = (
                            rows[bo + e, pl.ds(r * 16, 16)] * w)
                return carry

            lax.fori_loop(0, _K // 16, group, 0)

        def zrow(e, carry):
            for r in range(VR):
                rows[e, pl.ds(r * 16, 16)] = jnp.zeros((16,), jnp.float32)
            return carry

        lax.fori_loop(0, _NB * _K, zrow, 0)
        off = 0
        while off < _ROWS_PER_TILE:
            n = min(_NB * _K, _ROWS_PER_TILE - off)
            pltpu.sync_copy(rows.at[pl.ds(0, n)],
                            acc.at[pl.ds(s * _ROWS_PER_TILE + off, n)])
            off += n
        plsc.subcore_barrier()

        tile_chunk0 = (c * _NS + s) * _CHT

        for q in range(_NQ):
            pltpu.sync_copy(idx_hbm.at[pl.ds(tile_chunk0 + q * _QC, _QC)],
                            idxall)
            fire_gather(0, 0)
            fire_gather(_K, 1)

            def chunk(jj, carry):
                bo = lax.rem(jj, _NB) * _K
                boa = lax.rem(jj + 2, _NB) * _K
                wait_gather(bo, jj)
                scale(bo, jj)
                fire_scatter(bo, jj)

                @pl.when(jj + 2 < _QC)
                def _():
                    @pl.when(jj > 0)
                    def _():
                        wait_scatter_any()

                    fire_gather(boa, jj + 2)

                return carry

            lax.fori_loop(0, _QC, chunk, 0)
            wait_scatter_any()
            wait_scatter_any()
            wait_scatter_any()

        plsc.subcore_barrier()
        pltpu.sync_copy(acc.at[pl.ds(s * _ROWS_PER_TILE, _ROWS_PER_TILE)],
                        out.at[c, pl.ds(s * _ROWS_PER_TILE, _ROWS_PER_TILE)])

    return spmm_kernel


def _l1_body(x_ref, w1_ref, degt_ref, h1p_ref, dinv_ref):
    deg = degt_ref[:, 0:1] + degt_ref[:, 1:2] + 1.0
    dinv = lax.rsqrt(deg)
    h = jnp.dot(x_ref[...], w1_ref[...],
                preferred_element_type=jnp.float32,
                precision=lax.Precision.HIGHEST)
    h1p_ref[...] = h * dinv
    dinv_ref[...] = dinv


def _l2_body(p1_ref, h1p_ref, dinv_ref, b1_ref, w2_ref, h2p_ref):
    dinv = dinv_ref[...]
    ssum = p1_ref[0] + p1_ref[1] + h1p_ref[...]
    a = jnp.maximum(ssum * dinv + b1_ref[...], 0.0)
    h2 = jnp.dot(a, w2_ref[...],
                 preferred_element_type=jnp.float32,
                 precision=lax.Precision.HIGHEST)
    h2p_ref[...] = h2 * dinv


def _l3_body(p2_ref, h2p_ref, dinv_ref, b2_ref, wc_ref, bc_ref, out_ref):
    dinv = dinv_ref[...]
    ssum = p2_ref[0] + p2_ref[1] + h2p_ref[...]
    a = jnp.maximum(ssum * dinv + b2_ref[...], 0.0)
    out_ref[...] = jnp.dot(a, wc_ref[...],
                           preferred_element_type=jnp.float32,
                           precision=lax.Precision.HIGHEST) + bc_ref[...]


def kernel(x, edge_index, edge_weight, W1, b1, W2, b2, Wc, bc):
    _deg = _make_deg_kernel()
    _spmm = _make_spmm_kernel(HID)
    f32 = jnp.float32
    src = edge_index[0].astype(jnp.int32)
    dst = edge_index[1].astype(jnp.int32)
    ew = edge_weight.astype(f32)

    padn = EP - N_EDGES
    pad_ids = jnp.arange(padn, dtype=jnp.int32)
    src_p = jnp.concatenate([src, pad_ids % N_NODES]).reshape(EP // 128, 128)
    dst_p = jnp.concatenate([dst, pad_ids % NP]).reshape(EP // 128, 128)
    ew_f = jnp.concatenate([ew, jnp.zeros((padn,), f32)])
    ew_p = ew_f.reshape(EP // 128, 128)
    ew_bits = jax.lax.bitcast_convert_type(ew_f, jnp.int32)
    idx_pack = jnp.stack(
        [src_p.reshape(EP // _K, _K), dst_p.reshape(EP // _K, _K),
         ew_bits.reshape(EP // _K, _K)], axis=1)

    x_p = jnp.zeros((NP, IN_DIM), f32).at[:N_NODES].set(x.astype(f32))
    w2_p = jnp.zeros((HID, 128), f32).at[:, :HID2].set(W2.astype(f32))
    b2_p = jnp.zeros((128,), f32).at[:HID2].set(b2.astype(f32))
    wc_p = jnp.zeros((128, 128), f32).at[:HID2, :NUM_CLASSES].set(Wc.astype(f32))
    bc_p = jnp.zeros((128,), f32).at[:NUM_CLASSES].set(bc.astype(f32))

    degp = _deg(dst_p, ew_p)
    degt = degp.T

    h1p, dinv = pl.pallas_call(
        _l1_body,
        out_shape=(jax.ShapeDtypeStruct((NP, HID), f32),
                   jax.ShapeDtypeStruct((NP, 1), f32)),
    )(x_p, W1.astype(f32), degt)

    p1 = _spmm(idx_pack, h1p)

    h2p = pl.pallas_call(
        _l2_body,
        out_shape=jax.ShapeDtypeStruct((NP, 128), f32),
    )(p1, h1p, dinv, b1.astype(f32), w2_p)

    p2 = _spmm(idx_pack, h2p)

    out = pl.pallas_call(
        _l3_body,
        out_shape=jax.ShapeDtypeStruct((NP, 128), f32),
    )(p2, h2p, dinv, b2_p, wc_p, bc_p)

    return out[:N_NODES, :NUM_CLASSES]
